# Initial kernel scaffold; baseline (speedup 1.0000x reference)
import jax
import jax.numpy as jnp
from jax.experimental import pallas as pl


def kernel(x, edge_index, W_l1, b_l1, W_r1, W_l2, b_l2, W_r2):
    raise NotImplementedError("write your pallas kernel here")


# speedup vs baseline: 4.5617x; 4.5617x over previous
import functools

import jax
import jax.numpy as jnp
from jax import lax
from jax.experimental import pallas as pl
from jax.experimental.pallas import tpu as pltpu
from jax.experimental.pallas import tpu_sc as plsc

N = 10000
E = 320000
D = 128

NC = 2
NS = 16
NW = NC * NS
C = 128
CHUNKS = -(-E // (NW * C))
E_PAD = NW * C * CHUNKS
NP = 10240
RPT = NP // NS


def _sc_aggregate(z_hbm, src_hbm, dst_hbm, zeros_hbm, zeros_np_hbm,
                  ones_c_hbm, agg_out, deg_out, src_idx, dst_idx, rows_v,
                  deg_v, ones_v, sem, agg_sh, deg_sh, *, compute_deg):
    cid = lax.axis_index("c")
    sid = lax.axis_index("s")
    wid = cid * NS + sid

    pltpu.sync_copy(zeros_hbm, rows_v)
    for k in range(RPT // C):
        r = (sid * (RPT // C) + k) * C
        pltpu.sync_copy(rows_v, agg_sh.at[pl.ds(r, C)])
    if compute_deg:
        pltpu.sync_copy(zeros_np_hbm.at[pl.ds(sid * RPT, RPT)], deg_v)
        pltpu.sync_copy(deg_v, deg_sh.at[pl.ds(sid * RPT, RPT)])
        pltpu.sync_copy(ones_c_hbm, ones_v)
    plsc.subcore_barrier()

    base_w = wid * (CHUNKS * C)

    def body(i, carry):
        base = base_w + i * C
        pltpu.sync_copy(src_hbm.at[pl.ds(base, C)], src_idx)
        pltpu.sync_copy(dst_hbm.at[pl.ds(base, C)], dst_idx)
        pltpu.async_copy(z_hbm.at[src_idx], rows_v, sem).wait()
        pltpu.sync_copy(rows_v, agg_sh.at[dst_idx], add=True)
        if compute_deg:
            pltpu.sync_copy(ones_v, deg_sh.at[dst_idx], add=True)
        return carry

    lax.fori_loop(0, CHUNKS, body, 0)
    plsc.subcore_barrier()

    for k in range(RPT // C):
        r = (sid * (RPT // C) + k) * C
        pltpu.sync_copy(agg_sh.at[pl.ds(r, C)], rows_v)
        pltpu.sync_copy(rows_v, agg_out.at[cid, pl.ds(r, C)])
    if compute_deg:
        pltpu.sync_copy(deg_sh.at[pl.ds(sid * RPT, RPT)], deg_v)
        pltpu.sync_copy(deg_v, deg_out.at[cid, pl.ds(sid * RPT, RPT)])


@functools.lru_cache(maxsize=None)
def _make_sc_pass(compute_deg):
    mesh = plsc.VectorSubcoreMesh(core_axis_name="c", subcore_axis_name="s",
                                  num_cores=NC, num_subcores=NS)
    out_type = [jax.ShapeDtypeStruct((NC, NP, D), jnp.float32)]
    scratch = [
        pltpu.VMEM((C,), jnp.int32),
        pltpu.VMEM((C,), jnp.int32),
        pltpu.VMEM((C, D), jnp.float32),
        pltpu.VMEM((RPT,), jnp.float32),
        pltpu.VMEM((C,), jnp.float32),
        pltpu.SemaphoreType.DMA,
        pltpu.VMEM_SHARED((NP, D), jnp.float32),
        pltpu.VMEM_SHARED((NP,), jnp.float32),
    ]
    if compute_deg:
        out_type.append(jax.ShapeDtypeStruct((NC, NP), jnp.float32))
        body = functools.partial(_sc_aggregate, compute_deg=True)
    else:
        def body(z, s, d, z0, znp, o1, agg_out, *rest):
            _sc_aggregate(z, s, d, z0, znp, o1, agg_out, None, *rest,
                          compute_deg=False)
    return pl.kernel(body, out_type=out_type, mesh=mesh,
                     scratch_types=scratch)


def _tc_layer_body(agg_ref, deg_ref, z_ref, wl_ref, wr_ref, b_ref, out_ref):
    a = agg_ref[0] + agg_ref[1]
    dg = deg_ref[0] + deg_ref[1]
    inv = 1.0 / jnp.maximum(dg, 1.0)
    mean = a * inv[:, None]
    out = (jnp.dot(mean, wl_ref[...], preferred_element_type=jnp.float32)
           + jnp.dot(z_ref[...], wr_ref[...],
                     preferred_element_type=jnp.float32)
           + b_ref[...])
    out_ref[...] = jnp.maximum(out, 0.0)


BR = 2048


def _tc_layer(agg, deg, z, W_l, W_r, b):
    grid = (NP // BR,)
    return pl.pallas_call(
        _tc_layer_body,
        grid=grid,
        in_specs=[
            pl.BlockSpec((NC, BR, D), lambda i: (0, i, 0)),
            pl.BlockSpec((NC, BR), lambda i: (0, i)),
            pl.BlockSpec((BR, D), lambda i: (i, 0)),
            pl.BlockSpec((D, D), lambda i: (0, 0)),
            pl.BlockSpec((D, D), lambda i: (0, 0)),
            pl.BlockSpec((1, D), lambda i: (0, 0)),
        ],
        out_specs=pl.BlockSpec((BR, D), lambda i: (i, 0)),
        out_shape=jax.ShapeDtypeStruct((NP, D), jnp.float32),
    )(agg, deg, z, W_l, W_r, b)


@jax.jit
def kernel(x, edge_index, W_l1, b_l1, W_r1, W_l2, b_l2, W_r2):
    src = edge_index[0].astype(jnp.int32)
    dst = edge_index[1].astype(jnp.int32)
    src_p = jnp.concatenate(
        [src, jnp.zeros((E_PAD - E,), jnp.int32)])
    dst_p = jnp.concatenate(
        [dst, jnp.full((E_PAD - E,), N, jnp.int32)])
    x_p = jnp.pad(x, ((0, NP - N), (0, 0)))
    zeros = jnp.zeros((C, D), jnp.float32)
    zeros_np = jnp.zeros((NP,), jnp.float32)
    ones_c = jnp.ones((C,), jnp.float32)

    agg1, deg = _make_sc_pass(True)(x_p, src_p, dst_p, zeros, zeros_np,
                                    ones_c)
    h1 = _tc_layer(agg1, deg, x_p, W_l1, W_r1, b_l1.reshape(1, D))
    (agg2,) = _make_sc_pass(False)(h1, src_p, dst_p, zeros, zeros_np, ones_c)
    h2 = _tc_layer(agg2, deg, h1, W_l2, W_r2, b_l2.reshape(1, D))
    return h2[:N]
